# Initial kernel scaffold; baseline (speedup 1.0000x reference)
import jax
import jax.numpy as jnp
from jax.experimental import pallas as pl


def kernel(x, edge_index, W1, b1, W2, b2, W3, b3):
    raise NotImplementedError("write your pallas kernel here")


# speedup vs baseline: 23.9641x; 23.9641x over previous
import functools

import jax
import jax.numpy as jnp
from jax import lax
from jax.experimental import pallas as pl
from jax.experimental.pallas import tpu as pltpu
from jax.experimental.pallas import tpu_sc as plsc

N = 10000
E = 320000
IN_C = 128
OUT_C = 16

NC = 2
NS = 16
NW = NC * NS
CH = 128
EPT = 10240
EP = NW * EPT
NJ = EPT // CH
NP = 10112
RPT = NP // NS

def _deg_body(dst_hbm, z16_hbm, out_hbm, idx_v, ones_v, zbuf, acc, sem):
    c = lax.axis_index("c")
    s = lax.axis_index("s")
    wid = s * NC + c
    row = jnp.where(lax.iota(jnp.int32, 16) == 0, 1.0, 0.0)

    def mk(i, carry):
        ones_v[i, :] = row
        return carry

    lax.fori_loop(0, CH, mk, 0)
    pltpu.sync_copy(z16_hbm.at[pl.ds(s * RPT, RPT)], zbuf)
    pltpu.sync_copy(zbuf, acc.at[pl.ds(s * RPT, RPT)])
    pltpu.sync_copy(dst_hbm.at[pl.ds(wid * NJ, NJ)], idx_v)
    plsc.subcore_barrier()

    def body(j, carry):
        pltpu.sync_copy(ones_v, acc.at[idx_v.at[j]], add=True)
        return carry

    lax.fori_loop(0, NJ, body, 0)
    plsc.subcore_barrier()
    pltpu.sync_copy(acc.at[pl.ds(s * RPT, RPT)],
                    out_hbm.at[c, pl.ds(s * RPT, RPT)])


@functools.cache
def _deg_call():
    mesh = plsc.VectorSubcoreMesh(core_axis_name="c", subcore_axis_name="s",
                                  num_cores=NC, num_subcores=NS)
    return pl.kernel(
        _deg_body,
        out_type=jax.ShapeDtypeStruct((NC, NP, 16), jnp.float32),
        mesh=mesh,
        compiler_params=pltpu.CompilerParams(use_tc_tiling_on_sc=False),
        scratch_types=[
            pltpu.VMEM((NJ, CH), jnp.int32),
            pltpu.VMEM((CH, 16), jnp.float32),
            pltpu.VMEM((RPT, 16), jnp.float32),
            pltpu.VMEM_SHARED((NP, 16), jnp.float32),
            pltpu.SemaphoreType.DMA,
        ],
    )


def _prop_body(rows_hbm, six_hbm, dix_hbm, z32_hbm, out_hbm,
               sidx, didx, rbuf, zbuf, acc, sem):
    c = lax.axis_index("c")
    s = lax.axis_index("s")
    wid = s * NC + c
    pltpu.sync_copy(z32_hbm.at[pl.ds(s * RPT, RPT)], zbuf)
    pltpu.sync_copy(zbuf, acc.at[pl.ds(s * RPT, RPT)])
    pltpu.sync_copy(six_hbm.at[pl.ds(wid * NJ, NJ)], sidx)
    pltpu.sync_copy(dix_hbm.at[pl.ds(wid * NJ, NJ)], didx)
    plsc.subcore_barrier()

    def body(j, carry):
        pltpu.async_copy(rows_hbm.at[sidx.at[j]], rbuf, sem).wait()
        pltpu.sync_copy(rbuf, acc.at[didx.at[j]], add=True)
        return carry

    lax.fori_loop(0, NJ, body, 0)
    plsc.subcore_barrier()
    pltpu.sync_copy(acc.at[pl.ds(s * RPT, RPT)],
                    out_hbm.at[c, pl.ds(s * RPT, RPT)])


@functools.cache
def _prop_call():
    mesh = plsc.VectorSubcoreMesh(core_axis_name="c", subcore_axis_name="s",
                                  num_cores=NC, num_subcores=NS)
    return pl.kernel(
        _prop_body,
        out_type=jax.ShapeDtypeStruct((NC, NP, 32), jnp.float32),
        mesh=mesh,
        compiler_params=pltpu.CompilerParams(use_tc_tiling_on_sc=False),
        scratch_types=[
            pltpu.VMEM((NJ, CH), jnp.int32),
            pltpu.VMEM((NJ, CH), jnp.int32),
            pltpu.VMEM((CH, 32), jnp.float32),
            pltpu.VMEM((RPT, 32), jnp.float32),
            pltpu.VMEM_SHARED((NP, 32), jnp.float32),
            pltpu.SemaphoreType.DMA,
        ],
    )


def _mm1_body(x_ref, w_ref, o_ref):
    o_ref[...] = jnp.dot(x_ref[...], w_ref[...],
                         preferred_element_type=jnp.float32)


def _scale_body(cnt_ref, h0_ref, hs_ref, dinv_ref, idg_ref):
    deg = cnt_ref[0, :, 0:1] + cnt_ref[1, :, 0:1] + 1.0
    dinv = lax.rsqrt(deg)
    dinv_ref[...] = dinv
    idg_ref[...] = 1.0 / deg
    hs_ref[...] = h0_ref[...] * dinv


def _layer1_body(acc_ref, dinv_ref, idg_ref, h0_ref, b1_ref, w23_ref,
                 g_ref, gs_ref):
    ph = dinv_ref[...] * (acc_ref[0] + acc_ref[1]) \
        + h0_ref[...] * idg_ref[...] + b1_ref[...]
    h = jnp.maximum(ph, 0.0)
    g = jnp.dot(h, w23_ref[...], preferred_element_type=jnp.float32)
    g_ref[...] = g
    gs_ref[...] = g * dinv_ref[...]


def _final_body(acc_ref, dinv_ref, idg_ref, g_ref, b23_ref, eps_ref, z_ref):
    pg = dinv_ref[...] * (acc_ref[0] + acc_ref[1]) \
        + g_ref[...] * idg_ref[...] + b23_ref[...]
    mu = pg[:N, :OUT_C]
    ls = jnp.minimum(pg[:N, OUT_C:], 10.0)
    z_ref[...] = mu + eps_ref[...] * jnp.exp(ls)


def kernel(x, edge_index, W1, b1, W2, b2, W3, b3):
    f32 = jnp.float32
    src = edge_index[0]
    dst = edge_index[1]
    pad = EP - E
    src_p = jnp.concatenate([src, jnp.zeros((pad,), jnp.int32)])
    dst_p = jnp.concatenate([dst, jnp.full((pad,), N, jnp.int32)])
    six = src_p.reshape(NW * NJ, CH)
    dix = dst_p.reshape(NW * NJ, CH)
    x_p = jnp.pad(x, ((0, NP - N), (0, 0)))
    z16 = jnp.zeros((NP, 16), f32)
    z32 = jnp.zeros((NP, 32), f32)
    w23 = jnp.concatenate([W2, W3], axis=1)
    b23 = jnp.concatenate([b2, b3]).reshape(1, 32)
    b1r = b1.reshape(1, 32)
    eps = jax.random.normal(jax.random.key(42), (N, OUT_C), dtype=f32)

    counts = _deg_call()(dix, z16)

    h0 = pl.pallas_call(
        _mm1_body,
        out_shape=jax.ShapeDtypeStruct((NP, 32), f32),
    )(x_p, W1)

    hs, dinv, idg = pl.pallas_call(
        _scale_body,
        out_shape=[
            jax.ShapeDtypeStruct((NP, 32), f32),
            jax.ShapeDtypeStruct((NP, 1), f32),
            jax.ShapeDtypeStruct((NP, 1), f32),
        ],
    )(counts, h0)

    acc1 = _prop_call()(hs, six, dix, z32)

    g, gs = pl.pallas_call(
        _layer1_body,
        out_shape=[
            jax.ShapeDtypeStruct((NP, 32), f32),
            jax.ShapeDtypeStruct((NP, 32), f32),
        ],
    )(acc1, dinv, idg, h0, b1r, w23)

    acc2 = _prop_call()(gs, six, dix, z32)

    z = pl.pallas_call(
        _final_body,
        out_shape=jax.ShapeDtypeStruct((N, OUT_C), f32),
    )(acc2, dinv, idg, g, b23, eps)
    return z


# speedup vs baseline: 27.7515x; 1.1580x over previous
import functools

import jax
import jax.numpy as jnp
from jax import lax
from jax.experimental import pallas as pl
from jax.experimental.pallas import tpu as pltpu
from jax.experimental.pallas import tpu_sc as plsc

N = 10000
E = 320000
IN_C = 128
OUT_C = 16

NC = 2
NS = 16
NW = NC * NS
CH = 128
EPT = 10240
EP = NW * EPT
NJ = EPT // CH
NP = 10112
RPT = NP // NS

def _deg_body(dst_hbm, z16_hbm, out_hbm, idx_v, ones_v, zbuf, acc, sem):
    c = lax.axis_index("c")
    s = lax.axis_index("s")
    wid = s * NC + c
    row = jnp.where(lax.iota(jnp.int32, 16) == 0, 1.0, 0.0)

    def mk(i, carry):
        ones_v[i, :] = row
        return carry

    lax.fori_loop(0, CH, mk, 0)
    pltpu.sync_copy(z16_hbm.at[pl.ds(s * RPT, RPT)], zbuf)
    pltpu.sync_copy(zbuf, acc.at[pl.ds(s * RPT, RPT)])
    pltpu.sync_copy(dst_hbm.at[pl.ds(wid * NJ, NJ)], idx_v)
    plsc.subcore_barrier()

    def body(j, carry):
        pltpu.sync_copy(ones_v, acc.at[idx_v.at[j]], add=True)
        return carry

    lax.fori_loop(0, NJ, body, 0)
    plsc.subcore_barrier()
    pltpu.sync_copy(acc.at[pl.ds(s * RPT, RPT)],
                    out_hbm.at[c, pl.ds(s * RPT, RPT)])


@functools.cache
def _deg_call():
    mesh = plsc.VectorSubcoreMesh(core_axis_name="c", subcore_axis_name="s",
                                  num_cores=NC, num_subcores=NS)
    return pl.kernel(
        _deg_body,
        out_type=jax.ShapeDtypeStruct((NC, NP, 16), jnp.float32),
        mesh=mesh,
        compiler_params=pltpu.CompilerParams(use_tc_tiling_on_sc=False),
        scratch_types=[
            pltpu.VMEM((NJ, CH), jnp.int32),
            pltpu.VMEM((CH, 16), jnp.float32),
            pltpu.VMEM((RPT, 16), jnp.float32),
            pltpu.VMEM_SHARED((NP, 16), jnp.float32),
            pltpu.SemaphoreType.DMA,
        ],
    )


def _prop_body(rows_hbm, six_hbm, dix_hbm, z32_hbm, out_hbm,
               sidx, didx, rb0, rb1, zbuf, acc, gs0, gs1, ss0, ss1):
    c = lax.axis_index("c")
    s = lax.axis_index("s")
    wid = s * NC + c
    pltpu.sync_copy(six_hbm.at[pl.ds(wid * NJ, NJ)], sidx)
    pltpu.sync_copy(dix_hbm.at[pl.ds(wid * NJ, NJ)], didx)
    pltpu.sync_copy(z32_hbm.at[pl.ds(s * RPT, RPT)], zbuf)
    pltpu.sync_copy(zbuf, acc.at[pl.ds(s * RPT, RPT)])
    plsc.subcore_barrier()
    pltpu.async_copy(rows_hbm.at[sidx.at[0]], rb0, gs0)
    pltpu.async_copy(rows_hbm.at[sidx.at[1]], rb1, gs1)

    def body(i, carry):
        j0 = 2 * i
        pltpu.make_async_copy(rows_hbm.at[sidx.at[j0]], rb0, gs0).wait()
        pltpu.async_copy(rb0, acc.at[didx.at[j0]], ss0, add=True)
        pltpu.make_async_copy(rows_hbm.at[sidx.at[j0 + 1]], rb1, gs1).wait()
        pltpu.async_copy(rb1, acc.at[didx.at[j0 + 1]], ss1, add=True)

        @pl.when(i + 1 < NJ // 2)
        def _refill():
            pltpu.make_async_copy(rb0, acc.at[didx.at[j0]], ss0).wait()
            pltpu.async_copy(rows_hbm.at[sidx.at[j0 + 2]], rb0, gs0)
            pltpu.make_async_copy(rb1, acc.at[didx.at[j0 + 1]], ss1).wait()
            pltpu.async_copy(rows_hbm.at[sidx.at[j0 + 3]], rb1, gs1)

        return carry

    lax.fori_loop(0, NJ // 2, body, 0)
    pltpu.make_async_copy(rb0, acc.at[didx.at[NJ - 2]], ss0).wait()
    pltpu.make_async_copy(rb1, acc.at[didx.at[NJ - 1]], ss1).wait()
    plsc.subcore_barrier()
    pltpu.sync_copy(acc.at[pl.ds(s * RPT, RPT)],
                    out_hbm.at[c, pl.ds(s * RPT, RPT)])


@functools.cache
def _prop_call():
    mesh = plsc.VectorSubcoreMesh(core_axis_name="c", subcore_axis_name="s",
                                  num_cores=NC, num_subcores=NS)
    return pl.kernel(
        _prop_body,
        out_type=jax.ShapeDtypeStruct((NC, NP, 32), jnp.float32),
        mesh=mesh,
        compiler_params=pltpu.CompilerParams(use_tc_tiling_on_sc=False),
        scratch_types=[
            pltpu.VMEM((NJ, CH), jnp.int32),
            pltpu.VMEM((NJ, CH), jnp.int32),
            pltpu.VMEM((CH, 32), jnp.float32),
            pltpu.VMEM((CH, 32), jnp.float32),
            pltpu.VMEM((RPT, 32), jnp.float32),
            pltpu.VMEM_SHARED((NP, 32), jnp.float32),
            pltpu.SemaphoreType.DMA,
            pltpu.SemaphoreType.DMA,
            pltpu.SemaphoreType.DMA,
            pltpu.SemaphoreType.DMA,
        ],
    )


def _mm1_body(x_ref, w_ref, o_ref):
    o_ref[...] = jnp.dot(x_ref[...], w_ref[...],
                         preferred_element_type=jnp.float32)


def _scale_body(cnt_ref, h0_ref, hs_ref, dinv_ref, idg_ref):
    deg = cnt_ref[0, :, 0:1] + cnt_ref[1, :, 0:1] + 1.0
    dinv = lax.rsqrt(deg)
    dinv_ref[...] = dinv
    idg_ref[...] = 1.0 / deg
    hs_ref[...] = h0_ref[...] * dinv


def _layer1_body(acc_ref, dinv_ref, idg_ref, h0_ref, b1_ref, w23_ref,
                 g_ref, gs_ref):
    ph = dinv_ref[...] * (acc_ref[0] + acc_ref[1]) \
        + h0_ref[...] * idg_ref[...] + b1_ref[...]
    h = jnp.maximum(ph, 0.0)
    g = jnp.dot(h, w23_ref[...], preferred_element_type=jnp.float32)
    g_ref[...] = g
    gs_ref[...] = g * dinv_ref[...]


def _final_body(acc_ref, dinv_ref, idg_ref, g_ref, b23_ref, eps_ref, z_ref):
    pg = dinv_ref[...] * (acc_ref[0] + acc_ref[1]) \
        + g_ref[...] * idg_ref[...] + b23_ref[...]
    mu = pg[:N, :OUT_C]
    ls = jnp.minimum(pg[:N, OUT_C:], 10.0)
    z_ref[...] = mu + eps_ref[...] * jnp.exp(ls)


def kernel(x, edge_index, W1, b1, W2, b2, W3, b3):
    f32 = jnp.float32
    src = edge_index[0]
    dst = edge_index[1]
    pad = EP - E
    src_p = jnp.concatenate([src, jnp.zeros((pad,), jnp.int32)])
    dst_p = jnp.concatenate([dst, jnp.full((pad,), N, jnp.int32)])
    six = src_p.reshape(NW * NJ, CH)
    dix = dst_p.reshape(NW * NJ, CH)
    x_p = jnp.pad(x, ((0, NP - N), (0, 0)))
    z16 = jnp.zeros((NP, 16), f32)
    z32 = jnp.zeros((NP, 32), f32)
    w23 = jnp.concatenate([W2, W3], axis=1)
    b23 = jnp.concatenate([b2, b3]).reshape(1, 32)
    b1r = b1.reshape(1, 32)
    eps = jax.random.normal(jax.random.key(42), (N, OUT_C), dtype=f32)

    counts = _deg_call()(dix, z16)

    h0 = pl.pallas_call(
        _mm1_body,
        out_shape=jax.ShapeDtypeStruct((NP, 32), f32),
    )(x_p, W1)

    hs, dinv, idg = pl.pallas_call(
        _scale_body,
        out_shape=[
            jax.ShapeDtypeStruct((NP, 32), f32),
            jax.ShapeDtypeStruct((NP, 1), f32),
            jax.ShapeDtypeStruct((NP, 1), f32),
        ],
    )(counts, h0)

    acc1 = _prop_call()(hs, six, dix, z32)

    g, gs = pl.pallas_call(
        _layer1_body,
        out_shape=[
            jax.ShapeDtypeStruct((NP, 32), f32),
            jax.ShapeDtypeStruct((NP, 32), f32),
        ],
    )(acc1, dinv, idg, h0, b1r, w23)

    acc2 = _prop_call()(gs, six, dix, z32)

    z = pl.pallas_call(
        _final_body,
        out_shape=jax.ShapeDtypeStruct((N, OUT_C), f32),
    )(acc2, dinv, idg, g, b23, eps)
    return z


# speedup vs baseline: 50.2769x; 1.8117x over previous
import functools

import jax
import jax.numpy as jnp
from jax import lax
from jax.experimental import pallas as pl
from jax.experimental.pallas import tpu as pltpu
from jax.experimental.pallas import tpu_sc as plsc

N = 10000
E = 320000
IN_C = 128
OUT_C = 16

NC = 2
NS = 16
NW = NC * NS
CH = 128
EPT = 10240
EP = NW * EPT
NJ = EPT // CH
NP = 10112
RPT = NP // NS

def _deg_body(dst_hbm, z16_hbm, out_hbm, idx_v, ones_v, zbuf, acc, sem):
    c = lax.axis_index("c")
    s = lax.axis_index("s")
    wid = s * NC + c
    row = jnp.where(lax.iota(jnp.int32, 16) == 0, 1.0, 0.0)

    def mk(i, carry):
        ones_v[i, :] = row
        return carry

    lax.fori_loop(0, CH, mk, 0)
    pltpu.sync_copy(z16_hbm.at[pl.ds(s * RPT, RPT)], zbuf)
    pltpu.sync_copy(zbuf, acc.at[pl.ds(s * RPT, RPT)])
    pltpu.sync_copy(dst_hbm.at[pl.ds(wid * NJ, NJ)], idx_v)
    plsc.subcore_barrier()

    def body(j, carry):
        pltpu.sync_copy(ones_v, acc.at[idx_v.at[j]], add=True)
        return carry

    lax.fori_loop(0, NJ, body, 0)
    plsc.subcore_barrier()
    pltpu.sync_copy(acc.at[pl.ds(s * RPT, RPT)],
                    out_hbm.at[c, pl.ds(s * RPT, RPT)])


@functools.cache
def _deg_call():
    mesh = plsc.VectorSubcoreMesh(core_axis_name="c", subcore_axis_name="s",
                                  num_cores=NC, num_subcores=NS)
    return pl.kernel(
        _deg_body,
        out_type=jax.ShapeDtypeStruct((NC, NP, 16), jnp.float32),
        mesh=mesh,
        compiler_params=pltpu.CompilerParams(use_tc_tiling_on_sc=False),
        scratch_types=[
            pltpu.VMEM((NJ, CH), jnp.int32),
            pltpu.VMEM((CH, 16), jnp.float32),
            pltpu.VMEM((RPT, 16), jnp.float32),
            pltpu.VMEM_SHARED((NP, 16), jnp.float32),
            pltpu.SemaphoreType.DMA,
        ],
    )


def _prop_body(rows_hbm, six_hbm, dix_hbm, z32_hbm, out_hbm,
               sidx, didx, rb0, rb1, zbuf, acc, tbl, gs0, gs1, ss0, ss1):
    c = lax.axis_index("c")
    s = lax.axis_index("s")
    wid = s * NC + c
    pltpu.sync_copy(six_hbm.at[pl.ds(wid * NJ, NJ)], sidx)
    pltpu.sync_copy(dix_hbm.at[pl.ds(wid * NJ, NJ)], didx)
    rows_slc = pl.ds(s * RPT, RPT)
    pltpu.sync_copy(rows_hbm.at[rows_slc], zbuf)
    pltpu.sync_copy(zbuf, tbl.at[rows_slc])
    pltpu.sync_copy(z32_hbm.at[rows_slc], zbuf)
    pltpu.sync_copy(zbuf, acc.at[rows_slc])
    plsc.subcore_barrier()
    pltpu.async_copy(tbl.at[sidx.at[0]], rb0, gs0)
    pltpu.async_copy(tbl.at[sidx.at[1]], rb1, gs1)

    def body(i, carry):
        j0 = 2 * i
        pltpu.make_async_copy(tbl.at[sidx.at[j0]], rb0, gs0).wait()
        pltpu.async_copy(rb0, acc.at[didx.at[j0]], ss0, add=True)
        pltpu.make_async_copy(tbl.at[sidx.at[j0 + 1]], rb1, gs1).wait()
        pltpu.async_copy(rb1, acc.at[didx.at[j0 + 1]], ss1, add=True)

        @pl.when(i + 1 < NJ // 2)
        def _refill():
            pltpu.make_async_copy(rb0, acc.at[didx.at[j0]], ss0).wait()
            pltpu.async_copy(tbl.at[sidx.at[j0 + 2]], rb0, gs0)
            pltpu.make_async_copy(rb1, acc.at[didx.at[j0 + 1]], ss1).wait()
            pltpu.async_copy(tbl.at[sidx.at[j0 + 3]], rb1, gs1)

        return carry

    lax.fori_loop(0, NJ // 2, body, 0)
    pltpu.make_async_copy(rb0, acc.at[didx.at[NJ - 2]], ss0).wait()
    pltpu.make_async_copy(rb1, acc.at[didx.at[NJ - 1]], ss1).wait()
    plsc.subcore_barrier()
    pltpu.sync_copy(acc.at[rows_slc], out_hbm.at[c, rows_slc])


@functools.cache
def _prop_call():
    mesh = plsc.VectorSubcoreMesh(core_axis_name="c", subcore_axis_name="s",
                                  num_cores=NC, num_subcores=NS)
    return pl.kernel(
        _prop_body,
        out_type=jax.ShapeDtypeStruct((NC, NP, 32), jnp.float32),
        mesh=mesh,
        compiler_params=pltpu.CompilerParams(use_tc_tiling_on_sc=False),
        scratch_types=[
            pltpu.VMEM((NJ, CH), jnp.int32),
            pltpu.VMEM((NJ, CH), jnp.int32),
            pltpu.VMEM((CH, 32), jnp.float32),
            pltpu.VMEM((CH, 32), jnp.float32),
            pltpu.VMEM((RPT, 32), jnp.float32),
            pltpu.VMEM_SHARED((NP, 32), jnp.float32),
            pltpu.VMEM_SHARED((NP, 32), jnp.float32),
            pltpu.SemaphoreType.DMA,
            pltpu.SemaphoreType.DMA,
            pltpu.SemaphoreType.DMA,
            pltpu.SemaphoreType.DMA,
        ],
    )


def _mm1_body(x_ref, w_ref, o_ref):
    o_ref[...] = jnp.dot(x_ref[...], w_ref[...],
                         preferred_element_type=jnp.float32)


def _scale_body(cnt_ref, h0_ref, hs_ref, dinv_ref, idg_ref):
    deg = cnt_ref[0, :, 0:1] + cnt_ref[1, :, 0:1] + 1.0
    dinv = lax.rsqrt(deg)
    dinv_ref[...] = dinv
    idg_ref[...] = 1.0 / deg
    hs_ref[...] = h0_ref[...] * dinv


def _layer1_body(acc_ref, dinv_ref, idg_ref, h0_ref, b1_ref, w23_ref,
                 g_ref, gs_ref):
    ph = dinv_ref[...] * (acc_ref[0] + acc_ref[1]) \
        + h0_ref[...] * idg_ref[...] + b1_ref[...]
    h = jnp.maximum(ph, 0.0)
    g = jnp.dot(h, w23_ref[...], preferred_element_type=jnp.float32)
    g_ref[...] = g
    gs_ref[...] = g * dinv_ref[...]


def _final_body(acc_ref, dinv_ref, idg_ref, g_ref, b23_ref, eps_ref, z_ref):
    pg = dinv_ref[...] * (acc_ref[0] + acc_ref[1]) \
        + g_ref[...] * idg_ref[...] + b23_ref[...]
    mu = pg[:N, :OUT_C]
    ls = jnp.minimum(pg[:N, OUT_C:], 10.0)
    z_ref[...] = mu + eps_ref[...] * jnp.exp(ls)


def kernel(x, edge_index, W1, b1, W2, b2, W3, b3):
    f32 = jnp.float32
    src = edge_index[0]
    dst = edge_index[1]
    pad = EP - E
    src_p = jnp.concatenate([src, jnp.zeros((pad,), jnp.int32)])
    dst_p = jnp.concatenate([dst, jnp.full((pad,), N, jnp.int32)])
    six = src_p.reshape(NW * NJ, CH)
    dix = dst_p.reshape(NW * NJ, CH)
    x_p = jnp.pad(x, ((0, NP - N), (0, 0)))
    z16 = jnp.zeros((NP, 16), f32)
    z32 = jnp.zeros((NP, 32), f32)
    w23 = jnp.concatenate([W2, W3], axis=1)
    b23 = jnp.concatenate([b2, b3]).reshape(1, 32)
    b1r = b1.reshape(1, 32)
    eps = jax.random.normal(jax.random.key(42), (N, OUT_C), dtype=f32)

    counts = _deg_call()(dix, z16)

    h0 = pl.pallas_call(
        _mm1_body,
        out_shape=jax.ShapeDtypeStruct((NP, 32), f32),
    )(x_p, W1)

    hs, dinv, idg = pl.pallas_call(
        _scale_body,
        out_shape=[
            jax.ShapeDtypeStruct((NP, 32), f32),
            jax.ShapeDtypeStruct((NP, 1), f32),
            jax.ShapeDtypeStruct((NP, 1), f32),
        ],
    )(counts, h0)

    acc1 = _prop_call()(hs, six, dix, z32)

    g, gs = pl.pallas_call(
        _layer1_body,
        out_shape=[
            jax.ShapeDtypeStruct((NP, 32), f32),
            jax.ShapeDtypeStruct((NP, 32), f32),
        ],
    )(acc1, dinv, idg, h0, b1r, w23)

    acc2 = _prop_call()(gs, six, dix, z32)

    z = pl.pallas_call(
        _final_body,
        out_shape=jax.ShapeDtypeStruct((N, OUT_C), f32),
    )(acc2, dinv, idg, g, b23, eps)
    return z
